# baseline (speedup 1.0000x reference)
import functools

import jax
import jax.numpy as jnp
from jax import lax
from jax.experimental import pallas as pl
from jax.experimental.pallas import tpu as pltpu
from jax.experimental.pallas import tpu_sc as plsc

N = 10000
D = 128
E = 320000
L = 3

NC = 2
NS = 16
NW = NC * NS

GRP = 128
EPAD = ((E + NW * GRP * 8 - 1) // (NW * GRP * 8)) * (NW * GRP * 8)
NGRP = EPAD // GRP
GPW = NGRP // NW
NDUMMY = 240
NPAD = N + NDUMMY
RPT = NPAD // NS
BLK = 1024


def _zero_vmem_rows(ref, nrows, ncols):
    def row(i, _):
        for k in range(ncols // 16):
            ref[i, pl.ds(k * 16, 16)] = jnp.zeros((16,), jnp.float32)
        return 0
    lax.fori_loop(0, nrows, row, 0, unroll=False)


def _sc_mesh():
    return plsc.VectorSubcoreMesh(
        core_axis_name="c", subcore_axis_name="s", num_cores=NC, num_subcores=NS
    )


def _deg_body(dst_hbm, out_hbm, cnt_sh, idx_v, ones_v, bounce_v, dsem):
    c = lax.axis_index("c")
    s = lax.axis_index("s")
    wid = c * NS + s

    def ones_row(i, _):
        ones_v[i, pl.ds(0, 16)] = jnp.ones((16,), jnp.float32)
        return 0
    lax.fori_loop(0, GRP, ones_row, 0, unroll=False)
    _zero_vmem_rows(bounce_v, GRP, 16)

    base = s * RPT
    for t in range(RPT // GRP):
        pltpu.sync_copy(bounce_v, cnt_sh.at[pl.ds(base + t * GRP, GRP)])

    pltpu.sync_copy(dst_hbm.at[pl.ds(wid * GPW, GPW)], idx_v)
    plsc.subcore_barrier()

    def fire(j, _):
        pltpu.async_copy(ones_v, cnt_sh.at[idx_v.at[j]], dsem, add=True)
        return 0
    lax.fori_loop(0, GPW, fire, 0, unroll=False)

    def drain(j, _):
        pltpu.make_async_copy(ones_v, cnt_sh.at[idx_v.at[j]], dsem).wait()
        return 0
    lax.fori_loop(0, GPW, drain, 0, unroll=False)
    plsc.subcore_barrier()

    for t in range(RPT // GRP):
        pltpu.sync_copy(cnt_sh.at[pl.ds(base + t * GRP, GRP)], bounce_v)
        pltpu.sync_copy(bounce_v, out_hbm.at[c, pl.ds(base + t * GRP, GRP)])


def _degree_counts(dstp):
    fn = pl.kernel(
        _deg_body,
        out_type=jax.ShapeDtypeStruct((NC, NPAD, 16), jnp.float32),
        mesh=_sc_mesh(),
        scratch_types=[
            pltpu.VMEM_SHARED((NPAD, 16), jnp.float32),
            pltpu.VMEM((GPW, GRP), jnp.int32),
            pltpu.VMEM((GRP, 16), jnp.float32),
            pltpu.VMEM((GRP, 16), jnp.float32),
            pltpu.SemaphoreType.DMA,
        ],
    )
    return fn(dstp)


PGRP = 64
PNGRP = EPAD // PGRP
PGPW = PNGRP // NW
PICH = 16
PNCHK = PGPW // PICH
NBUF = 4
SKEW = 2


def _prop_body(g_hbm, src_hbm, dst_hbm, out_hbm, acc_sh, idx_s, idx_d, rows_v,
               gsem, ssem):
    c = lax.axis_index("c")
    s = lax.axis_index("s")
    wid = c * NS + s

    def gather(row, b):
        return pltpu.make_async_copy(g_hbm.at[idx_s.at[row]], rows_v.at[b],
                                     gsem.at[b])

    def scatter_fire(row, b):
        pltpu.async_copy(rows_v.at[b], acc_sh.at[idx_d.at[row]],
                         ssem.at[b], add=True)

    def scatter_wait(b):
        pltpu.make_async_copy(rows_v.at[b], acc_sh.at[idx_d.at[0]],
                              ssem.at[b]).wait()

    def zrow(i, _):
        for k in range(D // 16):
            rows_v[0, i, pl.ds(k * 16, 16)] = jnp.zeros((16,), jnp.float32)
        return 0
    lax.fori_loop(0, PGRP, zrow, 0, unroll=False)
    base = s * RPT

    def zcp(t):
        return pltpu.make_async_copy(
            rows_v.at[0], acc_sh.at[pl.ds(base + t * PGRP, PGRP)], gsem.at[0])
    for t in range(RPT // PGRP):
        zcp(t).start()
    for t in range(RPT // PGRP):
        zcp(t).wait()
    plsc.subcore_barrier()

    def chunk_body(k, _):
        p = lax.rem(k, 2) * PICH
        pp = PICH - p
        ebase = wid * PGPW + k * PICH
        pltpu.sync_copy(src_hbm.at[pl.ds(ebase, PICH)],
                        idx_s.at[pl.ds(p, PICH)])
        pltpu.sync_copy(dst_hbm.at[pl.ds(ebase, PICH)],
                        idx_d.at[pl.ds(p, PICH)])
        for i in range(PICH):
            b = i % NBUF
            if i < NBUF:
                @pl.when(k > 0)
                def _():
                    scatter_wait(b)
            else:
                scatter_wait(b)
            gather(p + i, b).start()
            ib = i - SKEW
            bb = ib % NBUF
            rowb = p + ib if ib >= 0 else pp + PICH + ib
            if ib >= 0:
                gather(rowb, bb).wait()
                scatter_fire(rowb, bb)
            else:
                @pl.when(k > 0)
                def _():
                    gather(rowb, bb).wait()
                    scatter_fire(rowb, bb)
        return 0
    lax.fori_loop(0, PNCHK, chunk_body, 0, unroll=False)

    p_last = ((PNCHK - 1) % 2) * PICH
    for i in range(PICH - SKEW, PICH):
        b = i % NBUF
        gather(p_last + i, b).wait()
        scatter_fire(p_last + i, b)
    for b in range(NBUF):
        scatter_wait(b)
    plsc.subcore_barrier()

    pltpu.sync_copy(acc_sh.at[pl.ds(base, RPT)],
                    out_hbm.at[c, pl.ds(base, RPT)])


def _propagate(g, srcp, dstp):
    fn = pl.kernel(
        _prop_body,
        out_type=jax.ShapeDtypeStruct((NC, NPAD, D), jnp.float32),
        mesh=_sc_mesh(),
        scratch_types=[
            pltpu.VMEM_SHARED((NPAD, D), jnp.float32),
            pltpu.VMEM((2 * PICH, PGRP), jnp.int32),
            pltpu.VMEM((2 * PICH, PGRP), jnp.int32),
            pltpu.VMEM((NBUF, PGRP, D), jnp.float32),
            pltpu.SemaphoreType.DMA((NBUF,)),
            pltpu.SemaphoreType.DMA((NBUF,)),
        ],
    )
    return fn(g, srcp, dstp)


def _prep_body(cnt_ref, x_ref, a0_ref, dinv_ref, g_ref, res_ref):
    cnt = cnt_ref[...]
    deg = 1.0 + cnt[0, :, 0] + cnt[1, :, 0]
    dinv = lax.rsqrt(deg)[:, None]
    x = x_ref[...]
    dinv_ref[...] = dinv
    g_ref[...] = x * dinv
    res_ref[...] = x * a0_ref[0, 0]


def _prep(cnt, x_pad, a0):
    return pl.pallas_call(
        _prep_body,
        grid=(NPAD // BLK,),
        in_specs=[
            pl.BlockSpec((NC, BLK, 16), lambda i: (0, i, 0)),
            pl.BlockSpec((BLK, D), lambda i: (i, 0)),
            pl.BlockSpec(memory_space=pltpu.SMEM),
        ],
        out_specs=[
            pl.BlockSpec((BLK, 1), lambda i: (i, 0)),
            pl.BlockSpec((BLK, D), lambda i: (i, 0)),
            pl.BlockSpec((BLK, D), lambda i: (i, 0)),
        ],
        out_shape=[
            jax.ShapeDtypeStruct((NPAD, 1), jnp.float32),
            jax.ShapeDtypeStruct((NPAD, D), jnp.float32),
            jax.ShapeDtypeStruct((NPAD, D), jnp.float32),
        ],
    )(cnt, x_pad, a0)


def _layer_body(s_ref, g_ref, dinv_ref, res_ref, w_ref, b_ref, a_ref,
                g_out_ref, res_out_ref):
    dinv = dinv_ref[...]
    t = (s_ref[0] + s_ref[1] + g_ref[...]) * dinv
    h = jnp.dot(t, w_ref[...], preferred_element_type=jnp.float32) + b_ref[...]
    res_out_ref[...] = res_ref[...] + a_ref[0, 0] * h
    g_out_ref[...] = h * dinv


def _layer(sacc, g, dinv, res, w, b, a):
    return pl.pallas_call(
        _layer_body,
        grid=(NPAD // BLK,),
        in_specs=[
            pl.BlockSpec((NC, BLK, D), lambda i: (0, i, 0)),
            pl.BlockSpec((BLK, D), lambda i: (i, 0)),
            pl.BlockSpec((BLK, 1), lambda i: (i, 0)),
            pl.BlockSpec((BLK, D), lambda i: (i, 0)),
            pl.BlockSpec((D, D), lambda i: (0, 0)),
            pl.BlockSpec((1, D), lambda i: (0, 0)),
            pl.BlockSpec(memory_space=pltpu.SMEM),
        ],
        out_specs=[
            pl.BlockSpec((BLK, D), lambda i: (i, 0)),
            pl.BlockSpec((BLK, D), lambda i: (i, 0)),
        ],
        out_shape=[
            jax.ShapeDtypeStruct((NPAD, D), jnp.float32),
            jax.ShapeDtypeStruct((NPAD, D), jnp.float32),
        ],
    )(sacc, g, dinv, res, w, b, a)


def kernel(x, edge_index, W0, W1, W2, b0, b1, b2, alphas):
    src = edge_index[0]
    dst = edge_index[1]
    pad_ids = (N + (jnp.arange(EPAD - E, dtype=jnp.int32) % NDUMMY))
    src_flat = jnp.concatenate([src, pad_ids])
    dst_flat = jnp.concatenate([dst, pad_ids])
    srcp = src_flat.reshape(PNGRP, PGRP)
    dstp = dst_flat.reshape(PNGRP, PGRP)
    x_pad = jnp.pad(x, ((0, NDUMMY), (0, 0)))

    cnt = _degree_counts(dst_flat.reshape(NGRP, GRP))
    dinv, g, res = _prep(cnt, x_pad, alphas[0].reshape(1, 1))

    for i, (w, b) in enumerate(((W0, b0), (W1, b1), (W2, b2))):
        s = _propagate(g, srcp, dstp)
        g, res = _layer(s, g, dinv, res, w, b.reshape(1, D),
                        alphas[i + 1].reshape(1, 1))
    return res[:N]


# speedup vs baseline: 1.0771x; 1.0771x over previous
import functools

import jax
import jax.numpy as jnp
from jax import lax
from jax.experimental import pallas as pl
from jax.experimental.pallas import tpu as pltpu
from jax.experimental.pallas import tpu_sc as plsc

N = 10000
D = 128
E = 320000
L = 3

NC = 2
NS = 16
NW = NC * NS

GRP = 128
EPAD = ((E + NW * GRP * 8 - 1) // (NW * GRP * 8)) * (NW * GRP * 8)
NGRP = EPAD // GRP
GPW = NGRP // NW
NDUMMY = 240
NPAD = N + NDUMMY
RPT = NPAD // NS
BLK = 2048


def _zero_vmem_rows(ref, nrows, ncols):
    def row(i, _):
        for k in range(ncols // 16):
            ref[i, pl.ds(k * 16, 16)] = jnp.zeros((16,), jnp.float32)
        return 0
    lax.fori_loop(0, nrows, row, 0, unroll=False)


def _sc_mesh():
    return plsc.VectorSubcoreMesh(
        core_axis_name="c", subcore_axis_name="s", num_cores=NC, num_subcores=NS
    )


def _deg_body(dst_hbm, out_hbm, cnt_sh, idx_v, ones_v, bounce_v, dsem):
    c = lax.axis_index("c")
    s = lax.axis_index("s")
    wid = c * NS + s

    def ones_row(i, _):
        ones_v[i, pl.ds(0, 16)] = jnp.ones((16,), jnp.float32)
        return 0
    lax.fori_loop(0, GRP, ones_row, 0, unroll=False)
    _zero_vmem_rows(bounce_v, GRP, 16)

    base = s * RPT
    for t in range(RPT // GRP):
        pltpu.sync_copy(bounce_v, cnt_sh.at[pl.ds(base + t * GRP, GRP)])

    pltpu.sync_copy(dst_hbm.at[pl.ds(wid * GPW, GPW)], idx_v)
    plsc.subcore_barrier()

    def fire(j, _):
        pltpu.async_copy(ones_v, cnt_sh.at[idx_v.at[j]], dsem, add=True)
        return 0
    lax.fori_loop(0, GPW, fire, 0, unroll=False)

    def drain(j, _):
        pltpu.make_async_copy(ones_v, cnt_sh.at[idx_v.at[j]], dsem).wait()
        return 0
    lax.fori_loop(0, GPW, drain, 0, unroll=False)
    plsc.subcore_barrier()

    for t in range(RPT // GRP):
        pltpu.sync_copy(cnt_sh.at[pl.ds(base + t * GRP, GRP)], bounce_v)
        pltpu.sync_copy(bounce_v, out_hbm.at[c, pl.ds(base + t * GRP, GRP)])


def _degree_counts(dstp):
    fn = pl.kernel(
        _deg_body,
        out_type=jax.ShapeDtypeStruct((NC, NPAD, 16), jnp.float32),
        mesh=_sc_mesh(),
        scratch_types=[
            pltpu.VMEM_SHARED((NPAD, 16), jnp.float32),
            pltpu.VMEM((GPW, GRP), jnp.int32),
            pltpu.VMEM((GRP, 16), jnp.float32),
            pltpu.VMEM((GRP, 16), jnp.float32),
            pltpu.SemaphoreType.DMA,
        ],
    )
    return fn(dstp)


PGRP = 64
PNGRP = EPAD // PGRP
PGPW = PNGRP // NW
PICH = 16
PNCHK = PGPW // PICH
NBUF = 4
SKEW = 2


def _prop_body(g_hbm, src_hbm, dst_hbm, out_hbm, acc_sh, idx_s, idx_d, rows_v,
               gsem, ssem, isem_s, isem_d):
    c = lax.axis_index("c")
    s = lax.axis_index("s")
    wid = c * NS + s

    def gather(row, b):
        return pltpu.make_async_copy(g_hbm.at[idx_s.at[row]], rows_v.at[b],
                                     gsem.at[b])

    def scatter_fire(row, b):
        pltpu.async_copy(rows_v.at[b], acc_sh.at[idx_d.at[row]],
                         ssem.at[b], add=True)

    def scatter_wait(b):
        pltpu.make_async_copy(rows_v.at[b], acc_sh.at[idx_d.at[0]],
                              ssem.at[b]).wait()

    def zrow(i, _):
        for k in range(D // 16):
            rows_v[0, i, pl.ds(k * 16, 16)] = jnp.zeros((16,), jnp.float32)
        return 0
    lax.fori_loop(0, PGRP, zrow, 0, unroll=False)
    base = s * RPT

    def zcp(t):
        return pltpu.make_async_copy(
            rows_v.at[0], acc_sh.at[pl.ds(base + t * PGRP, PGRP)], gsem.at[0])
    for t in range(RPT // PGRP):
        zcp(t).start()
    pltpu.sync_copy(src_hbm.at[pl.ds(wid * PGPW, PICH)],
                    idx_s.at[pl.ds(0, PICH)])
    pltpu.sync_copy(dst_hbm.at[pl.ds(wid * PGPW, PICH)],
                    idx_d.at[pl.ds(0, PICH)])
    for t in range(RPT // PGRP):
        zcp(t).wait()
    plsc.subcore_barrier()

    def fetch(k):
        p = lax.rem(k, 2) * PICH
        ebase = wid * PGPW + k * PICH
        return (pltpu.make_async_copy(src_hbm.at[pl.ds(ebase, PICH)],
                                      idx_s.at[pl.ds(p, PICH)], isem_s),
                pltpu.make_async_copy(dst_hbm.at[pl.ds(ebase, PICH)],
                                      idx_d.at[pl.ds(p, PICH)], isem_d))

    def chunk_body(k, _):
        p = lax.rem(k, 2) * PICH
        pp = PICH - p

        @pl.when(k > 0)
        def _():
            fs, fd = fetch(k)
            fs.wait()
            fd.wait()
        for i in range(PICH):
            if i == NBUF:
                @pl.when(k + 1 < PNCHK)
                def _():
                    fs, fd = fetch(k + 1)
                    fs.start()
                    fd.start()
            b = i % NBUF
            if i < NBUF:
                @pl.when(k > 0)
                def _():
                    scatter_wait(b)
            else:
                scatter_wait(b)
            gather(p + i, b).start()
            ib = i - SKEW
            bb = ib % NBUF
            rowb = p + ib if ib >= 0 else pp + PICH + ib
            if ib >= 0:
                gather(rowb, bb).wait()
                scatter_fire(rowb, bb)
            else:
                @pl.when(k > 0)
                def _():
                    gather(rowb, bb).wait()
                    scatter_fire(rowb, bb)
        return 0
    lax.fori_loop(0, PNCHK, chunk_body, 0, unroll=False)

    p_last = ((PNCHK - 1) % 2) * PICH
    for i in range(PICH - SKEW, PICH):
        b = i % NBUF
        gather(p_last + i, b).wait()
        scatter_fire(p_last + i, b)
    for b in range(NBUF):
        scatter_wait(b)
    plsc.subcore_barrier()

    pltpu.sync_copy(acc_sh.at[pl.ds(base, RPT)],
                    out_hbm.at[c, pl.ds(base, RPT)])


def _propagate(g, srcp, dstp):
    fn = pl.kernel(
        _prop_body,
        out_type=jax.ShapeDtypeStruct((NC, NPAD, D), jnp.float32),
        mesh=_sc_mesh(),
        scratch_types=[
            pltpu.VMEM_SHARED((NPAD, D), jnp.float32),
            pltpu.VMEM((2 * PICH, PGRP), jnp.int32),
            pltpu.VMEM((2 * PICH, PGRP), jnp.int32),
            pltpu.VMEM((NBUF, PGRP, D), jnp.float32),
            pltpu.SemaphoreType.DMA((NBUF,)),
            pltpu.SemaphoreType.DMA((NBUF,)),
            pltpu.SemaphoreType.DMA,
            pltpu.SemaphoreType.DMA,
        ],
    )
    return fn(g, srcp, dstp)


def _prep_body(cnt_ref, x_ref, a0_ref, dinv_ref, g_ref, res_ref):
    cnt = cnt_ref[...]
    deg = 1.0 + cnt[0, :, 0] + cnt[1, :, 0]
    dinv = lax.rsqrt(deg)[:, None]
    x = x_ref[...]
    dinv_ref[...] = dinv
    g_ref[...] = x * dinv
    res_ref[...] = x * a0_ref[0, 0]


def _prep(cnt, x_pad, a0):
    return pl.pallas_call(
        _prep_body,
        grid=(NPAD // BLK,),
        in_specs=[
            pl.BlockSpec((NC, BLK, 16), lambda i: (0, i, 0)),
            pl.BlockSpec((BLK, D), lambda i: (i, 0)),
            pl.BlockSpec(memory_space=pltpu.SMEM),
        ],
        out_specs=[
            pl.BlockSpec((BLK, 1), lambda i: (i, 0)),
            pl.BlockSpec((BLK, D), lambda i: (i, 0)),
            pl.BlockSpec((BLK, D), lambda i: (i, 0)),
        ],
        out_shape=[
            jax.ShapeDtypeStruct((NPAD, 1), jnp.float32),
            jax.ShapeDtypeStruct((NPAD, D), jnp.float32),
            jax.ShapeDtypeStruct((NPAD, D), jnp.float32),
        ],
    )(cnt, x_pad, a0)


def _layer_body(s_ref, g_ref, dinv_ref, res_ref, w_ref, b_ref, a_ref,
                g_out_ref, res_out_ref):
    dinv = dinv_ref[...]
    t = (s_ref[0] + s_ref[1] + g_ref[...]) * dinv
    h = jnp.dot(t, w_ref[...], preferred_element_type=jnp.float32) + b_ref[...]
    res_out_ref[...] = res_ref[...] + a_ref[0, 0] * h
    g_out_ref[...] = h * dinv


def _layer(sacc, g, dinv, res, w, b, a):
    return pl.pallas_call(
        _layer_body,
        grid=(NPAD // BLK,),
        in_specs=[
            pl.BlockSpec((NC, BLK, D), lambda i: (0, i, 0)),
            pl.BlockSpec((BLK, D), lambda i: (i, 0)),
            pl.BlockSpec((BLK, 1), lambda i: (i, 0)),
            pl.BlockSpec((BLK, D), lambda i: (i, 0)),
            pl.BlockSpec((D, D), lambda i: (0, 0)),
            pl.BlockSpec((1, D), lambda i: (0, 0)),
            pl.BlockSpec(memory_space=pltpu.SMEM),
        ],
        out_specs=[
            pl.BlockSpec((BLK, D), lambda i: (i, 0)),
            pl.BlockSpec((BLK, D), lambda i: (i, 0)),
        ],
        out_shape=[
            jax.ShapeDtypeStruct((NPAD, D), jnp.float32),
            jax.ShapeDtypeStruct((NPAD, D), jnp.float32),
        ],
    )(sacc, g, dinv, res, w, b, a)


def kernel(x, edge_index, W0, W1, W2, b0, b1, b2, alphas):
    src = edge_index[0]
    dst = edge_index[1]
    pad_ids = (N + (jnp.arange(EPAD - E, dtype=jnp.int32) % NDUMMY))
    src_flat = jnp.concatenate([src, pad_ids])
    dst_flat = jnp.concatenate([dst, pad_ids])
    srcp = src_flat.reshape(PNGRP, PGRP)
    dstp = dst_flat.reshape(PNGRP, PGRP)
    x_pad = jnp.pad(x, ((0, NDUMMY), (0, 0)))

    cnt = _degree_counts(dst_flat.reshape(NGRP, GRP))
    dinv, g, res = _prep(cnt, x_pad, alphas[0].reshape(1, 1))

    for i, (w, b) in enumerate(((W0, b0), (W1, b1), (W2, b2))):
        s = _propagate(g, srcp, dstp)
        g, res = _layer(s, g, dinv, res, w, b.reshape(1, D),
                        alphas[i + 1].reshape(1, 1))
    return res[:N]


# speedup vs baseline: 1.0917x; 1.0136x over previous
import functools

import jax
import jax.numpy as jnp
from jax import lax
from jax.experimental import pallas as pl
from jax.experimental.pallas import tpu as pltpu
from jax.experimental.pallas import tpu_sc as plsc

N = 10000
D = 128
E = 320000
L = 3

NC = 2
NS = 16
NW = NC * NS

GRP = 128
EPAD = ((E + NW * GRP * 8 - 1) // (NW * GRP * 8)) * (NW * GRP * 8)
NGRP = EPAD // GRP
GPW = NGRP // NW
NDUMMY = 240
NPAD = N + NDUMMY
RPT = NPAD // NS
BLK = 2048
BLK2 = 2000


def _zero_vmem_rows(ref, nrows, ncols):
    def row(i, _):
        for k in range(ncols // 16):
            ref[i, pl.ds(k * 16, 16)] = jnp.zeros((16,), jnp.float32)
        return 0
    lax.fori_loop(0, nrows, row, 0, unroll=False)


def _sc_mesh():
    return plsc.VectorSubcoreMesh(
        core_axis_name="c", subcore_axis_name="s", num_cores=NC, num_subcores=NS
    )


def _deg_body(dst_hbm, out_hbm, cnt_sh, idx_v, ones_v, bounce_v, dsem):
    c = lax.axis_index("c")
    s = lax.axis_index("s")
    wid = c * NS + s

    def ones_row(i, _):
        ones_v[i, pl.ds(0, 16)] = jnp.ones((16,), jnp.float32)
        return 0
    lax.fori_loop(0, GRP, ones_row, 0, unroll=False)
    _zero_vmem_rows(bounce_v, GRP, 16)

    base = s * RPT
    for t in range(RPT // GRP):
        pltpu.sync_copy(bounce_v, cnt_sh.at[pl.ds(base + t * GRP, GRP)])

    pltpu.sync_copy(dst_hbm.at[pl.ds(wid * GPW, GPW)], idx_v)
    plsc.subcore_barrier()

    def fire(j, _):
        pltpu.async_copy(ones_v, cnt_sh.at[idx_v.at[j]], dsem, add=True)
        return 0
    lax.fori_loop(0, GPW, fire, 0, unroll=False)

    def drain(j, _):
        pltpu.make_async_copy(ones_v, cnt_sh.at[idx_v.at[j]], dsem).wait()
        return 0
    lax.fori_loop(0, GPW, drain, 0, unroll=False)
    plsc.subcore_barrier()

    for t in range(RPT // GRP):
        pltpu.sync_copy(cnt_sh.at[pl.ds(base + t * GRP, GRP)], bounce_v)
        pltpu.sync_copy(bounce_v, out_hbm.at[c, pl.ds(base + t * GRP, GRP)])


def _degree_counts(dstp):
    fn = pl.kernel(
        _deg_body,
        out_type=jax.ShapeDtypeStruct((NC, NPAD, 16), jnp.float32),
        mesh=_sc_mesh(),
        scratch_types=[
            pltpu.VMEM_SHARED((NPAD, 16), jnp.float32),
            pltpu.VMEM((GPW, GRP), jnp.int32),
            pltpu.VMEM((GRP, 16), jnp.float32),
            pltpu.VMEM((GRP, 16), jnp.float32),
            pltpu.SemaphoreType.DMA,
        ],
    )
    return fn(dstp)


PGRP = 64
PNGRP = EPAD // PGRP
PGPW = PNGRP // NW
PICH = 16
PNCHK = PGPW // PICH
NBUF = 4
SKEW = 2


def _prop_body(g_hbm, src_hbm, dst_hbm, out_hbm, acc_sh, idx_s, idx_d, rows_v,
               gsem, ssem, isem_s, isem_d):
    c = lax.axis_index("c")
    s = lax.axis_index("s")
    wid = c * NS + s

    def gather(row, b):
        return pltpu.make_async_copy(g_hbm.at[idx_s.at[row]], rows_v.at[b],
                                     gsem.at[b])

    def scatter_fire(row, b):
        pltpu.async_copy(rows_v.at[b], acc_sh.at[idx_d.at[row]],
                         ssem.at[b], add=True)

    def scatter_wait(b):
        pltpu.make_async_copy(rows_v.at[b], acc_sh.at[idx_d.at[0]],
                              ssem.at[b]).wait()

    def zrow(i, _):
        for k in range(D // 16):
            rows_v[0, i, pl.ds(k * 16, 16)] = jnp.zeros((16,), jnp.float32)
        return 0
    lax.fori_loop(0, PGRP, zrow, 0, unroll=False)
    base = s * RPT

    def zcp(t):
        return pltpu.make_async_copy(
            rows_v.at[0], acc_sh.at[pl.ds(base + t * PGRP, PGRP)], gsem.at[0])
    for t in range(RPT // PGRP):
        zcp(t).start()
    pltpu.sync_copy(src_hbm.at[pl.ds(wid * PGPW, PICH)],
                    idx_s.at[pl.ds(0, PICH)])
    pltpu.sync_copy(dst_hbm.at[pl.ds(wid * PGPW, PICH)],
                    idx_d.at[pl.ds(0, PICH)])
    for t in range(RPT // PGRP):
        zcp(t).wait()
    plsc.subcore_barrier()

    def fetch(k):
        p = lax.rem(k, 2) * PICH
        ebase = wid * PGPW + k * PICH
        return (pltpu.make_async_copy(src_hbm.at[pl.ds(ebase, PICH)],
                                      idx_s.at[pl.ds(p, PICH)], isem_s),
                pltpu.make_async_copy(dst_hbm.at[pl.ds(ebase, PICH)],
                                      idx_d.at[pl.ds(p, PICH)], isem_d))

    def chunk_body(k, _):
        p = lax.rem(k, 2) * PICH
        pp = PICH - p

        @pl.when(k > 0)
        def _():
            fs, fd = fetch(k)
            fs.wait()
            fd.wait()
        for i in range(PICH):
            if i == NBUF:
                @pl.when(k + 1 < PNCHK)
                def _():
                    fs, fd = fetch(k + 1)
                    fs.start()
                    fd.start()
            b = i % NBUF
            if i < NBUF:
                @pl.when(k > 0)
                def _():
                    scatter_wait(b)
            else:
                scatter_wait(b)
            gather(p + i, b).start()
            ib = i - SKEW
            bb = ib % NBUF
            rowb = p + ib if ib >= 0 else pp + PICH + ib
            if ib >= 0:
                gather(rowb, bb).wait()
                scatter_fire(rowb, bb)
            else:
                @pl.when(k > 0)
                def _():
                    gather(rowb, bb).wait()
                    scatter_fire(rowb, bb)
        return 0
    lax.fori_loop(0, PNCHK, chunk_body, 0, unroll=False)

    p_last = ((PNCHK - 1) % 2) * PICH
    for i in range(PICH - SKEW, PICH):
        b = i % NBUF
        gather(p_last + i, b).wait()
        scatter_fire(p_last + i, b)
    for b in range(NBUF):
        scatter_wait(b)
    plsc.subcore_barrier()

    pltpu.sync_copy(acc_sh.at[pl.ds(base, RPT)],
                    out_hbm.at[c, pl.ds(base, RPT)])


def _propagate(g, srcp, dstp):
    fn = pl.kernel(
        _prop_body,
        out_type=jax.ShapeDtypeStruct((NC, NPAD, D), jnp.float32),
        mesh=_sc_mesh(),
        scratch_types=[
            pltpu.VMEM_SHARED((NPAD, D), jnp.float32),
            pltpu.VMEM((2 * PICH, PGRP), jnp.int32),
            pltpu.VMEM((2 * PICH, PGRP), jnp.int32),
            pltpu.VMEM((NBUF, PGRP, D), jnp.float32),
            pltpu.SemaphoreType.DMA((NBUF,)),
            pltpu.SemaphoreType.DMA((NBUF,)),
            pltpu.SemaphoreType.DMA,
            pltpu.SemaphoreType.DMA,
        ],
    )
    return fn(g, srcp, dstp)


def _prep_body(cnt_ref, x_ref, a0_ref, dinv_ref, g_ref, res_ref):
    cnt = cnt_ref[...]
    deg = 1.0 + cnt[0, :, 0] + cnt[1, :, 0]
    dinv = lax.rsqrt(deg)[:, None]
    x = x_ref[...]
    dinv_ref[...] = dinv
    g_ref[...] = x * dinv
    res_ref[...] = x * a0_ref[0, 0]


def _prep(cnt, x, a0):
    return pl.pallas_call(
        _prep_body,
        grid=(NPAD // BLK,),
        in_specs=[
            pl.BlockSpec((NC, BLK, 16), lambda i: (0, i, 0)),
            pl.BlockSpec((BLK, D), lambda i: (i, 0)),
            pl.BlockSpec(memory_space=pltpu.SMEM),
        ],
        out_specs=[
            pl.BlockSpec((BLK, 1), lambda i: (i, 0)),
            pl.BlockSpec((BLK, D), lambda i: (i, 0)),
            pl.BlockSpec((BLK, D), lambda i: (i, 0)),
        ],
        out_shape=[
            jax.ShapeDtypeStruct((NPAD, 1), jnp.float32),
            jax.ShapeDtypeStruct((NPAD, D), jnp.float32),
            jax.ShapeDtypeStruct((NPAD, D), jnp.float32),
        ],
    )(cnt, x, a0)


def _layer_body(s_ref, g_ref, dinv_ref, res_ref, w_ref, b_ref, a_ref,
                g_out_ref, res_out_ref):
    dinv = dinv_ref[...]
    t = (s_ref[0] + s_ref[1] + g_ref[...]) * dinv
    h = jnp.dot(t, w_ref[...], preferred_element_type=jnp.float32) + b_ref[...]
    res_out_ref[...] = res_ref[...] + a_ref[0, 0] * h
    g_out_ref[...] = h * dinv


def _last_body(s_ref, g_ref, dinv_ref, res_ref, w_ref, b_ref, a_ref,
               res_out_ref):
    t = (s_ref[0] + s_ref[1] + g_ref[...]) * dinv_ref[...]
    h = jnp.dot(t, w_ref[...], preferred_element_type=jnp.float32) + b_ref[...]
    res_out_ref[...] = res_ref[...] + a_ref[0, 0] * h


def _layer(sacc, g, dinv, res, w, b, a, last):
    blk = BLK2 if last else BLK
    nrow = N if last else NPAD
    in_specs = [
        pl.BlockSpec((NC, blk, D), lambda i: (0, i, 0)),
        pl.BlockSpec((blk, D), lambda i: (i, 0)),
        pl.BlockSpec((blk, 1), lambda i: (i, 0)),
        pl.BlockSpec((blk, D), lambda i: (i, 0)),
        pl.BlockSpec((D, D), lambda i: (0, 0)),
        pl.BlockSpec((1, D), lambda i: (0, 0)),
        pl.BlockSpec(memory_space=pltpu.SMEM),
    ]
    if last:
        return pl.pallas_call(
            _last_body,
            grid=(nrow // blk,),
            in_specs=in_specs,
            out_specs=pl.BlockSpec((blk, D), lambda i: (i, 0)),
            out_shape=jax.ShapeDtypeStruct((N, D), jnp.float32),
        )(sacc, g, dinv, res, w, b, a)
    return pl.pallas_call(
        _layer_body,
        grid=(nrow // blk,),
        in_specs=in_specs,
        out_specs=[
            pl.BlockSpec((blk, D), lambda i: (i, 0)),
            pl.BlockSpec((blk, D), lambda i: (i, 0)),
        ],
        out_shape=[
            jax.ShapeDtypeStruct((NPAD, D), jnp.float32),
            jax.ShapeDtypeStruct((NPAD, D), jnp.float32),
        ],
    )(sacc, g, dinv, res, w, b, a)


def kernel(x, edge_index, W0, W1, W2, b0, b1, b2, alphas):
    src = edge_index[0]
    dst = edge_index[1]
    pad_ids = (N + (jnp.arange(EPAD - E, dtype=jnp.int32) % NDUMMY))
    src_flat = jnp.concatenate([src, pad_ids])
    dst_flat = jnp.concatenate([dst, pad_ids])
    srcp = src_flat.reshape(PNGRP, PGRP)
    dstp = dst_flat.reshape(PNGRP, PGRP)
    x_pad = jnp.pad(x, ((0, NDUMMY), (0, 0)))
    cnt = _degree_counts(dst_flat.reshape(NGRP, GRP))
    dinv, g, res = _prep(cnt, x_pad, alphas[0].reshape(1, 1))

    for i, (w, b) in enumerate(((W0, b0), (W1, b1), (W2, b2))):
        s = _propagate(g, srcp, dstp)
        last = i == L - 1
        out = _layer(s, g, dinv, res, w, b.reshape(1, D),
                     alphas[i + 1].reshape(1, 1), last)
        if last:
            return out
        g, res = out
